# baseline (speedup 1.0000x reference)
import functools

import jax
import jax.numpy as jnp
from jax.experimental import pallas as pl
from jax.experimental.pallas import tpu as pltpu

_D_MODEL = 2048
_D_FF = 8192
_BM = 1024
_BF = 512

_NT = (((1,), (1,)), ((), ()))


def _ffn_block(x_ref, w1_ref, w2_ref, o_ref, xbf_ref):
    j = pl.program_id(1)

    @pl.when(j == 0)
    def _():
        xbf_ref[...] = x_ref[...].astype(jnp.bfloat16)

    w1 = w1_ref[...].astype(jnp.bfloat16)
    h = jax.lax.dot_general(xbf_ref[...], w1, _NT,
                            preferred_element_type=jnp.float32)
    g = jax.nn.gelu(h).astype(jnp.bfloat16)
    w2 = w2_ref[...].astype(jnp.bfloat16)
    p = jax.lax.dot_general(g, w2, _NT,
                            preferred_element_type=jnp.float32)

    @pl.when(j == 0)
    def _():
        o_ref[...] = p

    @pl.when(j != 0)
    def _():
        o_ref[...] += p


@functools.partial(jax.jit, static_argnums=())
def _ffn(x2d, W1, W2):
    m = x2d.shape[0]
    grid = (m // _BM, _D_FF // _BF)
    return pl.pallas_call(
        _ffn_block,
        grid=grid,
        in_specs=[
            pl.BlockSpec((_BM, _D_MODEL), lambda i, j: (i, 0)),
            pl.BlockSpec((_BF, _D_MODEL), lambda i, j: (j, 0)),
            pl.BlockSpec((_D_MODEL, _BF), lambda i, j: (0, j)),
        ],
        out_specs=pl.BlockSpec((_BM, _D_MODEL), lambda i, j: (i, 0)),
        out_shape=jax.ShapeDtypeStruct((m, _D_MODEL), jnp.float32),
        scratch_shapes=[pltpu.VMEM((_BM, _D_MODEL), jnp.bfloat16)],
        compiler_params=pltpu.CompilerParams(
            dimension_semantics=("arbitrary", "arbitrary"),
            vmem_limit_bytes=64 * 1024 * 1024,
        ),
    )(x2d, W1, W2)


def kernel(inputs, W1, W2):
    b, s, d = inputs.shape
    out = _ffn(inputs.reshape(b * s, d), W1, W2)
    return out.reshape(b, s, d)
